# Initial kernel scaffold; baseline (speedup 1.0000x reference)
import jax
import jax.numpy as jnp
from jax.experimental import pallas as pl


def kernel(hidden_states, Wq, Wk, Wv, Wo, Wg, Ck, Cv):
    raise NotImplementedError("write your pallas kernel here")


# speedup vs baseline: 1.6612x; 1.6612x over previous
import jax
import jax.numpy as jnp
from jax.experimental import pallas as pl
from jax.experimental.pallas import tpu as pltpu

S = 2048
HID = 768
H = 12
HK = 2
G = 6
D = 64
KS = 32
STR = 16
BS = 64
TOPK = 16
WIN = 512
THETA = 10000.0
T = 127
TP = 128
NB = 32
SCALE = 0.125
R = 256
KB = 256
NKB = S // KB
NEG = -1e30

f32 = jnp.float32


def _nn(a, b):
    return jax.lax.dot_general(a, b, (((1,), (0,)), ((), ())),
                               preferred_element_type=f32)


def _nt(a, b):
    return jax.lax.dot_general(a, b, (((1,), (1,)), ((), ())),
                               preferred_element_type=f32)


def _rope_heads(x, nh, cos, sin):
    outs = []
    for h in range(nh):
        x1 = x[:, h * D:h * D + D // 2]
        x2 = x[:, h * D + D // 2:(h + 1) * D]
        outs.append(x1 * cos - x2 * sin)
        outs.append(x2 * cos + x1 * sin)
    return jnp.concatenate(outs, axis=1)


def _proj_kernel(x_ref, wq_ref, wk_ref, wv_ref, wg_ref, cos_ref, sin_ref,
                 q_ref, k_ref, v_ref, g_ref):
    x = x_ref[:]
    cos = cos_ref[:]
    sin = sin_ref[:]
    q_ref[:] = _rope_heads(_nn(x, wq_ref[:]), H, cos, sin)
    k_ref[:] = _rope_heads(_nn(x, wk_ref[:]), HK, cos, sin)
    v_ref[:] = _nn(x, wv_ref[:])
    g_ref[:] = jax.nn.sigmoid(_nn(x, wg_ref[:]))


def _compress_kernel(k2_ref, v2_ref, wck_ref, wcv_ref, ck_ref, cv_ref):
    half = KS * D // 2
    zero = jnp.zeros((1, half), f32)
    for h in range(HK):
        a = k2_ref[h]
        b = v2_ref[h]
        a1 = jnp.concatenate([a[1:], zero], axis=0)
        b1 = jnp.concatenate([b[1:], zero], axis=0)
        ck_ref[h] = _nn(a, wck_ref[h, :half]) + _nn(a1, wck_ref[h, half:])
        cv_ref[h] = _nn(b, wcv_ref[h, :half]) + _nn(b1, wcv_ref[h, half:])


def _online_update(state, sij, mask, vb):
    m0, l0, a0 = state
    maskf = mask.astype(f32)
    sm = jnp.where(mask, sij, NEG)
    m1 = jnp.maximum(m0, jnp.max(sm, axis=1, keepdims=True))
    alpha = jnp.exp(m0 - m1)
    p = jnp.exp(sm - m1) * maskf
    l1 = l0 * alpha + jnp.sum(p, axis=1, keepdims=True)
    a1 = a0 * alpha + _nn(p, vb)
    return (m1, l1, a1)


def _attn_kernel(q_ref, k_ref, v_ref, ck_ref, cv_ref, g_ref, wo_ref,
                 o_ref, selx_ref):
    qb = pl.program_id(0)
    base = qb * R
    srow = base + jax.lax.broadcasted_iota(jnp.int32, (R, 1), 0)
    tcol = jax.lax.broadcasted_iota(jnp.int32, (R, TP), 1)
    allowed = (srow >= STR * tcol + KS - 1) & (tcol < T)
    allowf = allowed.astype(f32)
    psum = [jnp.zeros((R, TP), f32), jnp.zeros((R, TP), f32)]
    cmp_heads = []
    for hq in range(H):
        kvh = hq // G
        qh = q_ref[:, hq * D:(hq + 1) * D]
        sc = _nt(qh, ck_ref[kvh]) * SCALE
        scm = jnp.where(allowed, sc, NEG)
        m = jnp.max(scm, axis=1, keepdims=True)
        p = jnp.exp(scm - m) * allowf
        l = jnp.sum(p, axis=1, keepdims=True)
        pc = p / jnp.maximum(l, 1e-30)
        cmp_heads.append(_nn(pc, cv_ref[kvh]))
        psum[kvh] = psum[kvh] + pc
    t_r = jax.lax.broadcasted_iota(jnp.int32, (TP, NB), 0)
    n_c = jax.lax.broadcasted_iota(jnp.int32, (TP, NB), 1)
    agg = ((t_r // (BS // STR)) == n_c).astype(f32)
    qblk = srow // BS
    n_row = jax.lax.broadcasted_iota(jnp.int32, (R, NB), 1)
    forced = (n_row < 1) | ((n_row <= qblk) & (n_row >= qblk - 1))
    causal_b = n_row <= qblk
    for kvh in range(HK):
        blk = _nn(psum[kvh], agg)
        cand = jnp.where(forced, 1e9, blk)
        cand = jnp.where(causal_b, cand, NEG)
        gt = (cand[:, :, None] > cand[:, None, :]).astype(f32)
        cnt = jnp.sum(gt, axis=1)
        sel = ((cnt < float(TOPK)) & (cand > -1e29)).astype(f32)
        for kb in range(NKB):
            jc = jax.lax.broadcasted_iota(jnp.int32, (NB, KB), 1)
            nc2 = jax.lax.broadcasted_iota(jnp.int32, (NB, KB), 0)
            exp_kb = ((jc // BS + 4 * kb) == nc2).astype(f32)
            selx_ref[kvh, kb] = _nn(sel, exp_kb)
    rowi = jax.lax.broadcasted_iota(jnp.int32, (R, KB), 0)
    colj = jax.lax.broadcasted_iota(jnp.int32, (R, KB), 1)
    g = g_ref[:]
    g0, g1, g2 = g[:, 0:1], g[:, 1:2], g[:, 2:3]
    out_heads = []
    for hq in range(H):
        kvh = hq // G
        qh = q_ref[:, hq * D:(hq + 1) * D]

        def qk_vb(kb):
            kblk = k_ref[pl.ds(kb * KB, KB), kvh * D:(kvh + 1) * D]
            vblk = v_ref[pl.ds(kb * KB, KB), kvh * D:(kvh + 1) * D]
            return _nt(qh, kblk) * SCALE, vblk

        def body1(kb, st):
            sij, vb = qk_vb(kb)
            msk = selx_ref[kvh, kb] > 0.5
            return _online_update(st, sij, msk, vb)

        def body2(kb, carry):
            st_sp, st_sw = carry
            sij, vb = qk_vb(kb)
            p_abs = kb * KB + colj
            s_abs = base + rowi
            causal = p_abs <= s_abs
            msp = (selx_ref[kvh, kb] > 0.5) & causal
            msw = causal & ((s_abs - p_abs) <= WIN)
            return (_online_update(st_sp, sij, msp, vb),
                    _online_update(st_sw, sij, msw, vb))

        init = (jnp.full((R, 1), NEG, f32), jnp.zeros((R, 1), f32),
                jnp.zeros((R, D), f32))
        ub1 = jnp.maximum(qb - 2, 0)
        st_sp = jax.lax.fori_loop(0, ub1, body1, init)
        st_sp, st_sw = jax.lax.fori_loop(ub1, qb + 1, body2, (st_sp, init))
        o_sp = st_sp[2] / st_sp[1]
        o_sw = st_sw[2] / st_sw[1]
        out_heads.append(g0 * cmp_heads[hq] + g1 * o_sp + g2 * o_sw)
    combined = jnp.concatenate(out_heads, axis=1)
    o_ref[:] = _nn(combined, wo_ref[:])


def kernel(hidden_states, Wq, Wk, Wv, Wo, Wg, Ck, Cv):
    x = hidden_states[0]
    wq_t = Wq.T
    wk_t = Wk.T
    wv_t = Wv.T
    wg8 = jnp.zeros((8, HID), f32).at[:3].set(Wg)
    wg_t = wg8.T
    wo_t = Wo.T
    pos = jnp.arange(S, dtype=f32)
    inv = 1.0 / (THETA ** (jnp.arange(D // 2, dtype=f32) / (D // 2)))
    ang = pos[:, None] * inv[None, :]
    cos = jnp.cos(ang)
    sin = jnp.sin(ang)

    grid = S // R
    q, k, v, gate = pl.pallas_call(
        _proj_kernel,
        grid=(grid,),
        in_specs=[
            pl.BlockSpec((R, HID), lambda i: (i, 0)),
            pl.BlockSpec((HID, H * D), lambda i: (0, 0)),
            pl.BlockSpec((HID, HK * D), lambda i: (0, 0)),
            pl.BlockSpec((HID, HK * D), lambda i: (0, 0)),
            pl.BlockSpec((HID, 8), lambda i: (0, 0)),
            pl.BlockSpec((R, D // 2), lambda i: (i, 0)),
            pl.BlockSpec((R, D // 2), lambda i: (i, 0)),
        ],
        out_specs=[
            pl.BlockSpec((R, H * D), lambda i: (i, 0)),
            pl.BlockSpec((R, HK * D), lambda i: (i, 0)),
            pl.BlockSpec((R, HK * D), lambda i: (i, 0)),
            pl.BlockSpec((R, 8), lambda i: (i, 0)),
        ],
        out_shape=[
            jax.ShapeDtypeStruct((S, H * D), f32),
            jax.ShapeDtypeStruct((S, HK * D), f32),
            jax.ShapeDtypeStruct((S, HK * D), f32),
            jax.ShapeDtypeStruct((S, 8), f32),
        ],
    )(x, wq_t, wk_t, wv_t, wg_t, cos, sin)

    k2 = k.reshape(S // STR, STR, HK, D).transpose(2, 0, 1, 3).reshape(
        HK, S // STR, STR * D)
    v2 = v.reshape(S // STR, STR, HK, D).transpose(2, 0, 1, 3).reshape(
        HK, S // STR, STR * D)

    ck, cv = pl.pallas_call(
        _compress_kernel,
        grid=(1,),
        in_specs=[
            pl.BlockSpec((HK, S // STR, STR * D), lambda i: (0, 0, 0)),
            pl.BlockSpec((HK, S // STR, STR * D), lambda i: (0, 0, 0)),
            pl.BlockSpec((HK, KS * D, D), lambda i: (0, 0, 0)),
            pl.BlockSpec((HK, KS * D, D), lambda i: (0, 0, 0)),
        ],
        out_specs=[
            pl.BlockSpec((HK, TP, D), lambda i: (0, 0, 0)),
            pl.BlockSpec((HK, TP, D), lambda i: (0, 0, 0)),
        ],
        out_shape=[
            jax.ShapeDtypeStruct((HK, TP, D), f32),
            jax.ShapeDtypeStruct((HK, TP, D), f32),
        ],
    )(k2, v2, Ck, Cv)

    out = pl.pallas_call(
        _attn_kernel,
        grid=(grid,),
        in_specs=[
            pl.BlockSpec((R, H * D), lambda i: (i, 0)),
            pl.BlockSpec((S, HK * D), lambda i: (0, 0)),
            pl.BlockSpec((S, HK * D), lambda i: (0, 0)),
            pl.BlockSpec((HK, TP, D), lambda i: (0, 0, 0)),
            pl.BlockSpec((HK, TP, D), lambda i: (0, 0, 0)),
            pl.BlockSpec((R, 8), lambda i: (i, 0)),
            pl.BlockSpec((HID, HID), lambda i: (0, 0)),
        ],
        out_specs=pl.BlockSpec((R, HID), lambda i: (i, 0)),
        out_shape=jax.ShapeDtypeStruct((S, HID), f32),
        scratch_shapes=[pltpu.VMEM((HK, NKB, R, KB), f32)],
    )(q, k, v, ck, cv, gate, wo_t)

    return out[None]
